# Initial kernel scaffold; baseline (speedup 1.0000x reference)
import jax
import jax.numpy as jnp
from jax.experimental import pallas as pl


def kernel(X, U, V, prototype_bias, W_ctx, b_ctx, W_pre, b_pre):
    raise NotImplementedError("write your pallas kernel here")


# speedup vs baseline: 1.3386x; 1.3386x over previous
import math

import jax
import jax.numpy as jnp
from jax.experimental import pallas as pl
from jax.experimental.pallas import tpu as pltpu

E_HE_C = 4096
SPARSE_C = 0.0625
H_C = 12

TILE_N = 256


def _logits_body(x_ref, wpt_ref, bp_ref, mt_ref, out_ref, *, scale):
    x = x_ref[0]
    xp = jnp.dot(x, wpt_ref[...], preferred_element_type=jnp.float32)
    xp = xp + bp_ref[...][None, :]
    lg = jnp.dot(xp, mt_ref[0], preferred_element_type=jnp.float32)
    out_ref[0] = lg * scale


def kernel(X, U, V, prototype_bias, W_ctx, b_ctx, W_pre, b_pre):
    B, N, D = X.shape
    E, R = U.shape
    DH = D // H_C
    kE = max(1, int(E * SPARSE_C))
    scale = 1.0 / (math.sqrt(DH) * H_C)

    avg = X.mean(axis=1)
    mx = X.max(axis=1)
    context_cat = jnp.concatenate([avg, mx], axis=-1)
    V_offset = (context_cat @ W_ctx.T + b_ctx).reshape(B, R, D)
    V_dyn = V[None, :, :] + V_offset
    M = jnp.einsum('er,brd->bed', U, V_dyn) + prototype_bias[None]
    M_T = M.transpose(0, 2, 1)
    W_preT = W_pre.T

    grid = (B, N // TILE_N)
    logits = pl.pallas_call(
        lambda x, wpt, bp, mt, o: _logits_body(x, wpt, bp, mt, o, scale=scale),
        grid=grid,
        in_specs=[
            pl.BlockSpec((1, TILE_N, D), lambda b, i: (b, i, 0)),
            pl.BlockSpec((D, D), lambda b, i: (0, 0)),
            pl.BlockSpec((D,), lambda b, i: (0,)),
            pl.BlockSpec((1, D, E), lambda b, i: (b, 0, 0)),
        ],
        out_specs=pl.BlockSpec((1, TILE_N, E), lambda b, i: (b, i, 0)),
        out_shape=jax.ShapeDtypeStruct((B, N, E), jnp.float32),
    )(X, W_preT, b_pre, M_T)

    topv, topi = jax.lax.top_k(logits, kE)
    edge_w = jax.nn.softmax(topv, axis=-1)
    return (topi.astype(jnp.int32), edge_w)


# speedup vs baseline: 5.1275x; 3.8306x over previous
import functools
import math

import jax
import jax.numpy as jnp
from jax import lax
from jax.experimental import pallas as pl
from jax.experimental.pallas import tpu as pltpu
from jax.experimental.pallas import tpu_sc as plsc

E_C = 4096
SPARSE_C = 0.0625
H_C = 12

TILE_N = 256
K_C = 256
CAP = 512
NVR = CAP // 16
KVR = K_C // 16
C_THRESH = 1.3186

NC, NS, L = 2, 16, 16
NW = NC * NS
RB = 8


def _logits_body(x_ref, wpt_ref, bp_ref, mt_ref, out_ref, thr_ref, *, scale):
    x = x_ref[0]
    xp = jnp.dot(x, wpt_ref[...], preferred_element_type=jnp.float32)
    xp = xp + bp_ref[...][None, :]
    lg = jnp.dot(xp, mt_ref[0], preferred_element_type=jnp.float32) * scale
    out_ref[...] = lg
    mu = jnp.mean(lg, axis=-1)
    m2 = jnp.mean(lg * lg, axis=-1)
    sig = jnp.sqrt(jnp.maximum(m2 - mu * mu, 0.0))
    t = mu + C_THRESH * sig
    thr_ref[...] = jnp.broadcast_to(t[None, :], (8, t.shape[0]))


def _sc_body(lg_hbm, thr_hbm, idx_hbm, w_hbm, cnt_hbm,
             rowblk, thrw, cv, ci, outv, outi, cntb):
    rows = lg_hbm.shape[0]
    E = lg_hbm.shape[1]
    nv = E // 16
    rpw = rows // NW
    nblk = rpw // RB
    wid = lax.axis_index("s") * NC + lax.axis_index("c")
    base = wid * rpw
    pltpu.sync_copy(thr_hbm.at[pl.ds(base, rpw)], thrw)

    def blk_body(jb, _):
        row0 = base + jb * RB
        pltpu.sync_copy(lg_hbm.at[pl.ds(row0, RB), :], rowblk)

        def row_body(r, _):
            rl = jb * RB + r
            tb = thrw[pl.ds((rl // 16) * 16, 16)]
            tvec = tb.at[jnp.full((16,), rl % 16, jnp.int32)].get(
                mode="promise_in_bounds")

            for j in range(NVR + 1):
                cv[pl.ds(j * 16, 16)] = jnp.full((16,), -jnp.inf, jnp.float32)
                ci[pl.ds(j * 16, 16)] = jnp.zeros((16,), jnp.int32)

            def cbody(i, carry):
                off, cnt = carry
                v = rowblk[r, pl.ds(i * 16, 16)]
                m = v > tvec
                c = jnp.sum(m.astype(jnp.int32))

                @pl.when(c > 0)
                def _():
                    ii = lax.iota(jnp.int32, 16) + i * 16
                    plsc.store_compressed(cv.at[pl.ds(off, 16)], v, mask=m)
                    plsc.store_compressed(ci.at[pl.ds(off, 16)], ii, mask=m)

                return (jnp.minimum(off + c, CAP), cnt + c)

            _, count = lax.fori_loop(0, nv, cbody,
                                     (jnp.int32(0), jnp.int32(0)))

            for j in range(NVR):
                sk, sv = plsc.sort_key_val(cv[pl.ds(j * 16, 16)],
                                           ci[pl.ds(j * 16, 16)],
                                           descending=True)
                cv[pl.ds(j * 16, 16)] = sk
                ci[pl.ds(j * 16, 16)] = sv

            def clean(a0, m, top_only=False):
                s = m // 2
                while s >= 1:
                    for g0 in range(0, m, 2 * s):
                        for q in range(s):
                            xa, xb = a0 + g0 + q, a0 + g0 + q + s
                            av = cv[pl.ds(xa * 16, 16)]
                            ai = ci[pl.ds(xa * 16, 16)]
                            bv = cv[pl.ds(xb * 16, 16)]
                            bi = ci[pl.ds(xb * 16, 16)]
                            mk = av >= bv
                            cv[pl.ds(xa * 16, 16)] = jnp.where(mk, av, bv)
                            ci[pl.ds(xa * 16, 16)] = jnp.where(mk, ai, bi)
                            cv[pl.ds(xb * 16, 16)] = jnp.where(mk, bv, av)
                            ci[pl.ds(xb * 16, 16)] = jnp.where(mk, bi, ai)
                    s //= 2
                lim = m // 2 if top_only else m
                for q in range(lim):
                    sk, sv = plsc.sort_key_val(cv[pl.ds((a0 + q) * 16, 16)],
                                               ci[pl.ds((a0 + q) * 16, 16)],
                                               descending=True)
                    cv[pl.ds((a0 + q) * 16, 16)] = sk
                    ci[pl.ds((a0 + q) * 16, 16)] = sv

            m = 1
            while m < NVR:
                last = (2 * m == NVR)
                for a0 in range(0, NVR, 2 * m):
                    for q in range(m):
                        xa, xb = a0 + q, a0 + 2 * m - 1 - q
                        av = cv[pl.ds(xa * 16, 16)]
                        ai = ci[pl.ds(xa * 16, 16)]
                        bv = lax.rev(cv[pl.ds(xb * 16, 16)], (0,))
                        bi = lax.rev(ci[pl.ds(xb * 16, 16)], (0,))
                        mk = av >= bv
                        cv[pl.ds(xa * 16, 16)] = jnp.where(mk, av, bv)
                        ci[pl.ds(xa * 16, 16)] = jnp.where(mk, ai, bi)
                        lo_v = jnp.where(mk, bv, av)
                        lo_i = jnp.where(mk, bi, ai)
                        cv[pl.ds(xb * 16, 16)] = lax.rev(lo_v, (0,))
                        ci[pl.ds(xb * 16, 16)] = lax.rev(lo_i, (0,))
                    clean(a0, m)
                    if not last:
                        clean(a0 + m, m)
                m *= 2

            mx = cv[pl.ds(0, 16)].at[jnp.zeros((16,), jnp.int32)].get(
                mode="promise_in_bounds")
            es = []
            tot = jnp.float32(0.0)
            for j in range(KVR):
                e = jnp.exp(cv[pl.ds(j * 16, 16)] - mx)
                es.append(e)
                tot = tot + jnp.sum(e)
            totv = jnp.full((16,), tot, jnp.float32)
            for j in range(KVR):
                outv[pl.ds(r * K_C + j * 16, 16)] = es[j] / totv
                outi[pl.ds(r * K_C + j * 16, 16)] = ci[pl.ds(j * 16, 16)]
            cntb[pl.ds(r * 16, 16)] = jnp.full((16,), count, jnp.int32)
            return 0

        lax.fori_loop(0, RB, row_body, 0)
        pltpu.sync_copy(outi, idx_hbm.at[pl.ds(row0 * K_C, RB * K_C)])
        pltpu.sync_copy(outv, w_hbm.at[pl.ds(row0 * K_C, RB * K_C)])
        pltpu.sync_copy(cntb, cnt_hbm.at[pl.ds(row0 * 16, RB * 16)])
        return 0

    lax.fori_loop(0, nblk, blk_body, 0)


def kernel(X, U, V, prototype_bias, W_ctx, b_ctx, W_pre, b_pre):
    B, N, D = X.shape
    E, R = U.shape
    DH = D // H_C
    kE = max(1, int(E * SPARSE_C))
    scale = 1.0 / (math.sqrt(DH) * H_C)
    rows = B * N

    avg = X.mean(axis=1)
    mx = X.max(axis=1)
    context_cat = jnp.concatenate([avg, mx], axis=-1)
    V_offset = (context_cat @ W_ctx.T + b_ctx).reshape(B, R, D)
    V_dyn = V[None, :, :] + V_offset
    M = jnp.einsum('er,brd->bed', U, V_dyn) + prototype_bias[None]
    M_T = M.transpose(0, 2, 1)
    W_preT = W_pre.T

    grid = (B, N // TILE_N)
    logits, thr = pl.pallas_call(
        functools.partial(_logits_body, scale=scale),
        grid=grid,
        in_specs=[
            pl.BlockSpec((1, TILE_N, D), lambda b, i: (b, i, 0)),
            pl.BlockSpec((D, D), lambda b, i: (0, 0)),
            pl.BlockSpec((D,), lambda b, i: (0,)),
            pl.BlockSpec((1, D, E), lambda b, i: (b, 0, 0)),
        ],
        out_specs=[
            pl.BlockSpec((TILE_N, E), lambda b, i: (b * (N // TILE_N) + i, 0)),
            pl.BlockSpec((8, TILE_N), lambda b, i: (b * (N // TILE_N) + i, 0)),
        ],
        out_shape=[
            jax.ShapeDtypeStruct((rows, E), jnp.float32),
            jax.ShapeDtypeStruct((rows // TILE_N * 8, TILE_N), jnp.float32),
        ],
    )(X, W_preT, b_pre, M_T)

    thr_flat = thr.reshape(rows // TILE_N, 8, TILE_N)[:, 0, :].reshape(rows)

    mesh = plsc.VectorSubcoreMesh(core_axis_name="c", subcore_axis_name="s")
    sc = pl.kernel(
        _sc_body,
        mesh=mesh,
        compiler_params=pltpu.CompilerParams(needs_layout_passes=False),
        out_type=[
            jax.ShapeDtypeStruct((rows * K_C,), jnp.int32),
            jax.ShapeDtypeStruct((rows * K_C,), jnp.float32),
            jax.ShapeDtypeStruct((rows * 16,), jnp.int32),
        ],
        scratch_types=[
            pltpu.VMEM((RB, E), jnp.float32),
            pltpu.VMEM((rows // NW,), jnp.float32),
            pltpu.VMEM((CAP + 16,), jnp.float32),
            pltpu.VMEM((CAP + 16,), jnp.int32),
            pltpu.VMEM((RB * K_C,), jnp.float32),
            pltpu.VMEM((RB * K_C,), jnp.int32),
            pltpu.VMEM((RB * 16,), jnp.int32),
        ],
    )
    idx_f, w_f, cnt_f = sc(logits, thr_flat)

    counts = cnt_f.reshape(rows, 16)[:, 0]
    bad = jnp.any((counts < kE) | (counts > CAP))

    def fb(_):
        tv, ti = jax.lax.top_k(logits.reshape(B, N, E), kE)
        return ti.astype(jnp.int32), jax.nn.softmax(tv, axis=-1)

    def ok(_):
        return (idx_f.reshape(B, N, kE), w_f.reshape(B, N, kE))

    return jax.lax.cond(bad, fb, ok, None)


# speedup vs baseline: 5.6860x; 1.1089x over previous
import functools
import math

import jax
import jax.numpy as jnp
from jax import lax
from jax.experimental import pallas as pl
from jax.experimental.pallas import tpu as pltpu
from jax.experimental.pallas import tpu_sc as plsc

E_C = 4096
SPARSE_C = 0.0625
H_C = 12

TILE_N = 256
K_C = 256
CAP = 512
NVR = CAP // 16
KVR = K_C // 16
C_THRESH = 1.3186

NC, NS, L = 2, 16, 16
NW = NC * NS
RB = 8


def _logits_body(x_ref, wpt_ref, bp_ref, mt_ref, out_ref, thr_ref, *, scale):
    x = x_ref[0]
    xp = jnp.dot(x, wpt_ref[...], preferred_element_type=jnp.float32)
    xp = xp + bp_ref[...][None, :]
    lg = jnp.dot(xp, mt_ref[0], preferred_element_type=jnp.float32) * scale
    out_ref[...] = lg
    mu = jnp.mean(lg, axis=-1)
    m2 = jnp.mean(lg * lg, axis=-1)
    sig = jnp.sqrt(jnp.maximum(m2 - mu * mu, 0.0))
    t = mu + C_THRESH * sig
    thr_ref[...] = jnp.broadcast_to(t[None, :], (8, t.shape[0]))


def _sc_body(lg_hbm, thr_hbm, idx_hbm, w_hbm, cnt_hbm,
             rowblk, thrw, cv, ci, outv, outi, cntb):
    rows = lg_hbm.shape[0]
    E = lg_hbm.shape[1]
    nv = E // 16
    rpw = rows // NW
    nblk = rpw // RB
    wid = lax.axis_index("s") * NC + lax.axis_index("c")
    base = wid * rpw
    pltpu.sync_copy(thr_hbm.at[pl.ds(base, rpw)], thrw)

    def blk_body(jb, _):
        row0 = base + jb * RB
        pltpu.sync_copy(lg_hbm.at[pl.ds(row0, RB), :], rowblk)

        def row_body(r, _):
            rl = jb * RB + r
            tb = thrw[pl.ds((rl // 16) * 16, 16)]
            tvec = tb.at[jnp.full((16,), rl % 16, jnp.int32)].get(
                mode="promise_in_bounds")

            for j in range(NVR + 1):
                cv[pl.ds(j * 16, 16)] = jnp.full((16,), -jnp.inf, jnp.float32)
                ci[pl.ds(j * 16, 16)] = jnp.zeros((16,), jnp.int32)

            def cbody(i, carry):
                off, cnt = carry
                v = rowblk[r, pl.ds(i * 16, 16)]
                m = v > tvec
                c = plsc.all_reduce_population_count(m)[0]

                @pl.when(c > 0)
                def _():
                    ii = lax.iota(jnp.int32, 16) + i * 16
                    plsc.store_compressed(cv.at[pl.ds(off, 16)], v, mask=m)
                    plsc.store_compressed(ci.at[pl.ds(off, 16)], ii, mask=m)

                return (jnp.minimum(off + c, CAP), cnt + c)

            _, count = lax.fori_loop(0, nv, cbody,
                                     (jnp.int32(0), jnp.int32(0)),
                                     unroll=4)

            for j in range(NVR):
                sk, sv = plsc.sort_key_val(cv[pl.ds(j * 16, 16)],
                                           ci[pl.ds(j * 16, 16)],
                                           descending=True)
                cv[pl.ds(j * 16, 16)] = sk
                ci[pl.ds(j * 16, 16)] = sv

            def clean(a0, m, top_only=False):
                s = m // 2
                while s >= 1:
                    for g0 in range(0, m, 2 * s):
                        for q in range(s):
                            xa, xb = a0 + g0 + q, a0 + g0 + q + s
                            av = cv[pl.ds(xa * 16, 16)]
                            ai = ci[pl.ds(xa * 16, 16)]
                            bv = cv[pl.ds(xb * 16, 16)]
                            bi = ci[pl.ds(xb * 16, 16)]
                            mk = av >= bv
                            cv[pl.ds(xa * 16, 16)] = jnp.where(mk, av, bv)
                            ci[pl.ds(xa * 16, 16)] = jnp.where(mk, ai, bi)
                            cv[pl.ds(xb * 16, 16)] = jnp.where(mk, bv, av)
                            ci[pl.ds(xb * 16, 16)] = jnp.where(mk, bi, ai)
                    s //= 2
                lim = m // 2 if top_only else m
                for q in range(lim):
                    sk, sv = plsc.sort_key_val(cv[pl.ds((a0 + q) * 16, 16)],
                                               ci[pl.ds((a0 + q) * 16, 16)],
                                               descending=True)
                    cv[pl.ds((a0 + q) * 16, 16)] = sk
                    ci[pl.ds((a0 + q) * 16, 16)] = sv

            m = 1
            while m < NVR:
                last = (2 * m == NVR)
                for a0 in range(0, NVR, 2 * m):
                    for q in range(m):
                        xa, xb = a0 + q, a0 + 2 * m - 1 - q
                        av = cv[pl.ds(xa * 16, 16)]
                        ai = ci[pl.ds(xa * 16, 16)]
                        bv = lax.rev(cv[pl.ds(xb * 16, 16)], (0,))
                        bi = lax.rev(ci[pl.ds(xb * 16, 16)], (0,))
                        mk = av >= bv
                        cv[pl.ds(xa * 16, 16)] = jnp.where(mk, av, bv)
                        ci[pl.ds(xa * 16, 16)] = jnp.where(mk, ai, bi)
                        lo_v = jnp.where(mk, bv, av)
                        lo_i = jnp.where(mk, bi, ai)
                        cv[pl.ds(xb * 16, 16)] = lax.rev(lo_v, (0,))
                        ci[pl.ds(xb * 16, 16)] = lax.rev(lo_i, (0,))
                    clean(a0, m)
                    if not last:
                        clean(a0 + m, m)
                m *= 2

            mx = cv[pl.ds(0, 16)].at[jnp.zeros((16,), jnp.int32)].get(
                mode="promise_in_bounds")
            es = []
            totv = jnp.zeros((16,), jnp.float32)
            for j in range(KVR):
                e = jnp.exp(cv[pl.ds(j * 16, 16)] - mx)
                es.append(e)
                totv = totv + e
            lane = lax.iota(jnp.int32, 16)
            for st in (8, 4, 2, 1):
                totv = totv + totv.at[lane ^ st].get(mode="promise_in_bounds")
            for j in range(KVR):
                outv[pl.ds(r * K_C + j * 16, 16)] = es[j] / totv
                outi[pl.ds(r * K_C + j * 16, 16)] = ci[pl.ds(j * 16, 16)]
            cntb[pl.ds(r * 16, 16)] = jnp.full((16,), count, jnp.int32)
            return 0

        lax.fori_loop(0, RB, row_body, 0)
        pltpu.sync_copy(outi, idx_hbm.at[pl.ds(row0 * K_C, RB * K_C)])
        pltpu.sync_copy(outv, w_hbm.at[pl.ds(row0 * K_C, RB * K_C)])
        pltpu.sync_copy(cntb, cnt_hbm.at[pl.ds(row0 * 16, RB * 16)])
        return 0

    lax.fori_loop(0, nblk, blk_body, 0)


def kernel(X, U, V, prototype_bias, W_ctx, b_ctx, W_pre, b_pre):
    B, N, D = X.shape
    E, R = U.shape
    DH = D // H_C
    kE = max(1, int(E * SPARSE_C))
    scale = 1.0 / (math.sqrt(DH) * H_C)
    rows = B * N

    avg = X.mean(axis=1)
    mx = X.max(axis=1)
    context_cat = jnp.concatenate([avg, mx], axis=-1)
    V_offset = (context_cat @ W_ctx.T + b_ctx).reshape(B, R, D)
    V_dyn = V[None, :, :] + V_offset
    M = jnp.einsum('er,brd->bed', U, V_dyn) + prototype_bias[None]
    M_T = M.transpose(0, 2, 1)
    W_preT = W_pre.T

    grid = (B, N // TILE_N)
    logits, thr = pl.pallas_call(
        functools.partial(_logits_body, scale=scale),
        grid=grid,
        in_specs=[
            pl.BlockSpec((1, TILE_N, D), lambda b, i: (b, i, 0)),
            pl.BlockSpec((D, D), lambda b, i: (0, 0)),
            pl.BlockSpec((D,), lambda b, i: (0,)),
            pl.BlockSpec((1, D, E), lambda b, i: (b, 0, 0)),
        ],
        out_specs=[
            pl.BlockSpec((TILE_N, E), lambda b, i: (b * (N // TILE_N) + i, 0)),
            pl.BlockSpec((8, TILE_N), lambda b, i: (b * (N // TILE_N) + i, 0)),
        ],
        out_shape=[
            jax.ShapeDtypeStruct((rows, E), jnp.float32),
            jax.ShapeDtypeStruct((rows // TILE_N * 8, TILE_N), jnp.float32),
        ],
    )(X, W_preT, b_pre, M_T)

    thr_flat = thr.reshape(rows // TILE_N, 8, TILE_N)[:, 0, :].reshape(rows)

    mesh = plsc.VectorSubcoreMesh(core_axis_name="c", subcore_axis_name="s")
    sc = pl.kernel(
        _sc_body,
        mesh=mesh,
        compiler_params=pltpu.CompilerParams(needs_layout_passes=False),
        out_type=[
            jax.ShapeDtypeStruct((rows * K_C,), jnp.int32),
            jax.ShapeDtypeStruct((rows * K_C,), jnp.float32),
            jax.ShapeDtypeStruct((rows * 16,), jnp.int32),
        ],
        scratch_types=[
            pltpu.VMEM((RB, E), jnp.float32),
            pltpu.VMEM((rows // NW,), jnp.float32),
            pltpu.VMEM((CAP + 16,), jnp.float32),
            pltpu.VMEM((CAP + 16,), jnp.int32),
            pltpu.VMEM((RB * K_C,), jnp.float32),
            pltpu.VMEM((RB * K_C,), jnp.int32),
            pltpu.VMEM((RB * 16,), jnp.int32),
        ],
    )
    idx_f, w_f, cnt_f = sc(logits, thr_flat)

    counts = cnt_f.reshape(rows, 16)[:, 0]
    bad = jnp.any((counts < kE) | (counts > CAP))

    def fb(_):
        tv, ti = jax.lax.top_k(logits.reshape(B, N, E), kE)
        return ti.astype(jnp.int32), jax.nn.softmax(tv, axis=-1)

    def ok(_):
        return (idx_f.reshape(B, N, kE), w_f.reshape(B, N, kE))

    return jax.lax.cond(bad, fb, ok, None)


# speedup vs baseline: 6.8236x; 1.2001x over previous
import functools
import math

import jax
import jax.numpy as jnp
from jax import lax
from jax.experimental import pallas as pl
from jax.experimental.pallas import tpu as pltpu
from jax.experimental.pallas import tpu_sc as plsc

E_C = 4096
SPARSE_C = 0.0625
H_C = 12

TILE_N = 256
K_C = 256
CAP = 512
NVR = CAP // 16
KVR = K_C // 16
C_THRESH = 1.3186

NC, NS, L = 2, 16, 16
NW = NC * NS
RB = 8
CAPP = CAP + 16


def _logits_body(x_ref, wpt_ref, bp_ref, mt_ref, out_ref, thr_ref, *, scale):
    x = x_ref[0]
    xp = jnp.dot(x, wpt_ref[...], preferred_element_type=jnp.float32)
    xp = xp + bp_ref[...][None, :]
    lg = jnp.dot(xp, mt_ref[0], preferred_element_type=jnp.float32) * scale
    out_ref[...] = lg
    mu = jnp.mean(lg, axis=-1)
    m2 = jnp.mean(lg * lg, axis=-1)
    sig = jnp.sqrt(jnp.maximum(m2 - mu * mu, 0.0))
    t = mu + C_THRESH * sig
    thr_ref[...] = jnp.broadcast_to(t[None, :], (8, t.shape[0]))


def _sc_body(lg_hbm, thr_hbm, idx_hbm, w_hbm, cnt_hbm,
             rowg, thrw, cvg, cig, outv, outi, cntw):
    rows, E = lg_hbm.shape
    rpw = rows // NW
    ngrp = rpw // 16
    wid = lax.axis_index("s") * NC + lax.axis_index("c")
    base = wid * rpw
    pltpu.sync_copy(thr_hbm.at[pl.ds(base, rpw)], thrw)
    lane = lax.iota(jnp.int32, 16)
    neginf = jnp.full((16,), -jnp.inf, jnp.float32)

    def grp_body(jb, _):
        row0 = base + jb * 16
        pltpu.sync_copy(lg_hbm.at[pl.ds(row0, 16), :], rowg)
        tvec = thrw[pl.ds(jb * 16, 16)]

        def pf(j, _2):
            cvg[pl.ds(j * 16, 16)] = neginf
            return 0
        lax.fori_loop(0, 16 * CAPP // 16, pf, 0, unroll=8)

        rowoff = lane * CAPP
        def cbody(i, offs):
            pos = (lane + i) & (E - 1)
            v = plsc.load_gather(rowg, [lane, pos])
            m = v > tvec
            waddr = rowoff + jnp.minimum(offs, CAP)
            plsc.store_scatter(cvg, [waddr], v, mask=m)
            plsc.store_scatter(cig, [waddr], pos, mask=m)
            return offs + m.astype(jnp.int32)
        offs = lax.fori_loop(0, E, cbody, jnp.zeros((16,), jnp.int32),
                             unroll=8)
        cntw[pl.ds(jb * 16, 16)] = offs

        def row_body(r, _2):
            b0 = r * CAPP

            for j in range(NVR):
                sk, sv = plsc.sort_key_val(cvg[pl.ds(b0 + j * 16, 16)],
                                           cig[pl.ds(b0 + j * 16, 16)],
                                           descending=True)
                cvg[pl.ds(b0 + j * 16, 16)] = sk
                cig[pl.ds(b0 + j * 16, 16)] = sv

            def clean(a0, m):
                s = m // 2
                while s >= 1:
                    for g0 in range(0, m, 2 * s):
                        for q in range(s):
                            xa = b0 + (a0 + g0 + q) * 16
                            xb = b0 + (a0 + g0 + q + s) * 16
                            av = cvg[pl.ds(xa, 16)]
                            ai = cig[pl.ds(xa, 16)]
                            bv = cvg[pl.ds(xb, 16)]
                            bi = cig[pl.ds(xb, 16)]
                            mk = av >= bv
                            cvg[pl.ds(xa, 16)] = jnp.where(mk, av, bv)
                            cig[pl.ds(xa, 16)] = jnp.where(mk, ai, bi)
                            cvg[pl.ds(xb, 16)] = jnp.where(mk, bv, av)
                            cig[pl.ds(xb, 16)] = jnp.where(mk, bi, ai)
                    s //= 2
                for q in range(m):
                    xa = b0 + (a0 + q) * 16
                    sk, sv = plsc.sort_key_val(cvg[pl.ds(xa, 16)],
                                               cig[pl.ds(xa, 16)],
                                               descending=True)
                    cvg[pl.ds(xa, 16)] = sk
                    cig[pl.ds(xa, 16)] = sv

            m = 1
            while m < NVR:
                for a0 in range(0, NVR, 2 * m):
                    for q in range(m):
                        xa = b0 + (a0 + q) * 16
                        xb = b0 + (a0 + 2 * m - 1 - q) * 16
                        av = cvg[pl.ds(xa, 16)]
                        ai = cig[pl.ds(xa, 16)]
                        bv = lax.rev(cvg[pl.ds(xb, 16)], (0,))
                        bi = lax.rev(cig[pl.ds(xb, 16)], (0,))
                        mk = av >= bv
                        cvg[pl.ds(xa, 16)] = jnp.where(mk, av, bv)
                        cig[pl.ds(xa, 16)] = jnp.where(mk, ai, bi)
                        lo_v = jnp.where(mk, bv, av)
                        lo_i = jnp.where(mk, bi, ai)
                        cvg[pl.ds(xb, 16)] = lax.rev(lo_v, (0,))
                        cig[pl.ds(xb, 16)] = lax.rev(lo_i, (0,))
                    clean(a0, m)
                    if 2 * m != NVR:
                        clean(a0 + m, m)
                m *= 2

            mx = cvg[pl.ds(b0, 16)].at[jnp.zeros((16,), jnp.int32)].get(
                mode="promise_in_bounds")
            es = []
            totv = jnp.zeros((16,), jnp.float32)
            for j in range(KVR):
                e = jnp.exp(cvg[pl.ds(b0 + j * 16, 16)] - mx)
                es.append(e)
                totv = totv + e
            for st in (8, 4, 2, 1):
                totv = totv + totv.at[lane ^ st].get(mode="promise_in_bounds")
            for j in range(KVR):
                outv[pl.ds(r * K_C + j * 16, 16)] = es[j] / totv
                outi[pl.ds(r * K_C + j * 16, 16)] = cig[pl.ds(b0 + j * 16, 16)]
            return 0

        lax.fori_loop(0, 16, row_body, 0)
        pltpu.sync_copy(outi, idx_hbm.at[pl.ds(row0 * K_C, 16 * K_C)])
        pltpu.sync_copy(outv, w_hbm.at[pl.ds(row0 * K_C, 16 * K_C)])
        return 0

    lax.fori_loop(0, ngrp, grp_body, 0)
    pltpu.sync_copy(cntw, cnt_hbm.at[pl.ds(base, rpw)])


def kernel(X, U, V, prototype_bias, W_ctx, b_ctx, W_pre, b_pre):
    B, N, D = X.shape
    E, R = U.shape
    DH = D // H_C
    kE = max(1, int(E * SPARSE_C))
    scale = 1.0 / (math.sqrt(DH) * H_C)
    rows = B * N

    avg = X.mean(axis=1)
    mx = X.max(axis=1)
    context_cat = jnp.concatenate([avg, mx], axis=-1)
    V_offset = (context_cat @ W_ctx.T + b_ctx).reshape(B, R, D)
    V_dyn = V[None, :, :] + V_offset
    M = jnp.einsum('er,brd->bed', U, V_dyn) + prototype_bias[None]
    M_T = M.transpose(0, 2, 1)
    W_preT = W_pre.T

    grid = (B, N // TILE_N)
    logits, thr = pl.pallas_call(
        functools.partial(_logits_body, scale=scale),
        grid=grid,
        in_specs=[
            pl.BlockSpec((1, TILE_N, D), lambda b, i: (b, i, 0)),
            pl.BlockSpec((D, D), lambda b, i: (0, 0)),
            pl.BlockSpec((D,), lambda b, i: (0,)),
            pl.BlockSpec((1, D, E), lambda b, i: (b, 0, 0)),
        ],
        out_specs=[
            pl.BlockSpec((TILE_N, E), lambda b, i: (b * (N // TILE_N) + i, 0)),
            pl.BlockSpec((8, TILE_N), lambda b, i: (b * (N // TILE_N) + i, 0)),
        ],
        out_shape=[
            jax.ShapeDtypeStruct((rows, E), jnp.float32),
            jax.ShapeDtypeStruct((rows // TILE_N * 8, TILE_N), jnp.float32),
        ],
    )(X, W_preT, b_pre, M_T)

    thr_flat = thr.reshape(rows // TILE_N, 8, TILE_N)[:, 0, :].reshape(rows)

    mesh = plsc.VectorSubcoreMesh(core_axis_name="c", subcore_axis_name="s")
    sc = pl.kernel(
        _sc_body,
        mesh=mesh,
        compiler_params=pltpu.CompilerParams(needs_layout_passes=False),
        out_type=[
            jax.ShapeDtypeStruct((rows * K_C,), jnp.int32),
            jax.ShapeDtypeStruct((rows * K_C,), jnp.float32),
            jax.ShapeDtypeStruct((rows,), jnp.int32),
        ],
        scratch_types=[
            pltpu.VMEM((16, E), jnp.float32),
            pltpu.VMEM((rows // NW,), jnp.float32),
            pltpu.VMEM((16 * CAPP,), jnp.float32),
            pltpu.VMEM((16 * CAPP,), jnp.int32),
            pltpu.VMEM((16 * K_C,), jnp.float32),
            pltpu.VMEM((16 * K_C,), jnp.int32),
            pltpu.VMEM((rows // NW,), jnp.int32),
        ],
    )
    idx_f, w_f, cnt_f = sc(logits, thr_flat)

    counts = cnt_f
    bad = jnp.any((counts < kE) | (counts > CAP))

    def fb(_):
        tv, ti = jax.lax.top_k(logits.reshape(B, N, E), kE)
        return ti.astype(jnp.int32), jax.nn.softmax(tv, axis=-1)

    def ok(_):
        return (idx_f.reshape(B, N, kE), w_f.reshape(B, N, kE))

    return jax.lax.cond(bad, fb, ok, None)


# speedup vs baseline: 12.0915x; 1.7720x over previous
import functools
import math

import jax
import jax.numpy as jnp
from jax import lax
from jax.experimental import pallas as pl
from jax.experimental.pallas import tpu as pltpu
from jax.experimental.pallas import tpu_sc as plsc

E_C = 4096
SPARSE_C = 0.0625
H_C = 12

TILE_N = 256
K_C = 256
CAP = 512
NVR = CAP // 16
KVR = K_C // 16
C_THRESH = 1.3186

NC, NS, L = 2, 16, 16
NW = NC * NS
RB = 8
CAPP = CAP + 16


def _logits_body(x_ref, wpt_ref, bp_ref, mt_ref, out_ref, thr_ref, *, scale):
    x = x_ref[0]
    xp = jnp.dot(x, wpt_ref[...], preferred_element_type=jnp.float32)
    xp = xp + bp_ref[...][None, :]
    lg = jnp.dot(xp, mt_ref[0], preferred_element_type=jnp.float32) * scale
    out_ref[...] = lg
    mu = jnp.mean(lg, axis=-1)
    m2 = jnp.mean(lg * lg, axis=-1)
    sig = jnp.sqrt(jnp.maximum(m2 - mu * mu, 0.0))
    t = mu + C_THRESH * sig
    thr_ref[...] = jnp.broadcast_to(t[None, :], (8, t.shape[0]))


def _sc_body(lg_hbm, thr_hbm, idx_hbm, w_hbm, cnt_hbm,
             rowg, thrw, cvg, cig, outv, outi, cntw):
    rows, E = lg_hbm.shape
    rpw = rows // NW
    ngrp = rpw // 16
    wid = lax.axis_index("s") * NC + lax.axis_index("c")
    base = wid * rpw
    pltpu.sync_copy(thr_hbm.at[pl.ds(base, rpw)], thrw)
    lane = lax.iota(jnp.int32, 16)
    neginf = jnp.full((16,), -jnp.inf, jnp.float32)

    def grp_body(jb, _):
        row0 = base + jb * 16
        pltpu.sync_copy(lg_hbm.at[pl.ds(row0, 16), :], rowg)
        tvec = thrw[pl.ds(jb * 16, 16)]

        def pf(j, _2):
            cvg[pl.ds(j * 16, 16)] = neginf
            return 0
        lax.fori_loop(0, 16 * CAPP // 16, pf, 0, unroll=8)

        rowoff = lane * CAPP
        def cbody(i, offs):
            pos = (lane + i) & (E - 1)
            v = plsc.load_gather(rowg, [lane, pos])
            m = v > tvec
            waddr = rowoff + jnp.minimum(offs, CAP)
            plsc.store_scatter(cvg, [waddr], v, mask=m)
            plsc.store_scatter(cig, [waddr], pos, mask=m)
            return offs + m.astype(jnp.int32)
        offs = lax.fori_loop(0, E, cbody, jnp.zeros((16,), jnp.int32),
                             unroll=8)
        cntw[pl.ds(jb * 16, 16)] = offs

        def row_body(r, _2):
            b0 = r * CAPP

            va = []
            ia = []
            for j in range(NVR):
                sk, sv = plsc.sort_key_val(cvg[pl.ds(b0 + j * 16, 16)],
                                           cig[pl.ds(b0 + j * 16, 16)],
                                           descending=True)
                va.append(sk)
                ia.append(sv)

            def clean(a0, m):
                s2 = m // 2
                while s2 >= 1:
                    for g0 in range(0, m, 2 * s2):
                        for q in range(s2):
                            xa, xb = a0 + g0 + q, a0 + g0 + q + s2
                            mk = va[xa] >= va[xb]
                            hv = jnp.where(mk, va[xa], va[xb])
                            hi = jnp.where(mk, ia[xa], ia[xb])
                            lv = jnp.where(mk, va[xb], va[xa])
                            li = jnp.where(mk, ia[xb], ia[xa])
                            va[xa], ia[xa] = hv, hi
                            va[xb], ia[xb] = lv, li
                    s2 //= 2
                for q in range(m):
                    va[a0 + q], ia[a0 + q] = plsc.sort_key_val(
                        va[a0 + q], ia[a0 + q], descending=True)

            m = 1
            while m < NVR:
                last = (2 * m == NVR)
                for a0 in range(0, NVR, 2 * m):
                    for q in range(m):
                        xa, xb = a0 + q, a0 + 2 * m - 1 - q
                        bv = lax.rev(va[xb], (0,))
                        bi = lax.rev(ia[xb], (0,))
                        mk = va[xa] >= bv
                        hv = jnp.where(mk, va[xa], bv)
                        hi = jnp.where(mk, ia[xa], bi)
                        if not last:
                            lv = jnp.where(mk, bv, va[xa])
                            li = jnp.where(mk, bi, ia[xa])
                            va[xb] = lax.rev(lv, (0,))
                            ia[xb] = lax.rev(li, (0,))
                        va[xa], ia[xa] = hv, hi
                    clean(a0, m)
                    if not last:
                        clean(a0 + m, m)
                m *= 2

            mx = va[0].at[jnp.zeros((16,), jnp.int32)].get(
                mode="promise_in_bounds")
            es = []
            totv = jnp.zeros((16,), jnp.float32)
            for j in range(KVR):
                e = jnp.exp(va[j] - mx)
                es.append(e)
                totv = totv + e
            for st in (8, 4, 2, 1):
                totv = totv + totv.at[lane ^ st].get(mode="promise_in_bounds")
            for j in range(KVR):
                outv[pl.ds(r * K_C + j * 16, 16)] = es[j] / totv
                outi[pl.ds(r * K_C + j * 16, 16)] = ia[j]
            return 0

        lax.fori_loop(0, 16, row_body, 0)
        pltpu.sync_copy(outi, idx_hbm.at[pl.ds(row0 * K_C, 16 * K_C)])
        pltpu.sync_copy(outv, w_hbm.at[pl.ds(row0 * K_C, 16 * K_C)])
        return 0

    lax.fori_loop(0, ngrp, grp_body, 0)
    pltpu.sync_copy(cntw, cnt_hbm.at[pl.ds(base, rpw)])


def kernel(X, U, V, prototype_bias, W_ctx, b_ctx, W_pre, b_pre):
    B, N, D = X.shape
    E, R = U.shape
    DH = D // H_C
    kE = max(1, int(E * SPARSE_C))
    scale = 1.0 / (math.sqrt(DH) * H_C)
    rows = B * N

    avg = X.mean(axis=1)
    mx = X.max(axis=1)
    context_cat = jnp.concatenate([avg, mx], axis=-1)
    V_offset = (context_cat @ W_ctx.T + b_ctx).reshape(B, R, D)
    V_dyn = V[None, :, :] + V_offset
    M = jnp.einsum('er,brd->bed', U, V_dyn) + prototype_bias[None]
    M_T = M.transpose(0, 2, 1)
    W_preT = W_pre.T

    grid = (B, N // TILE_N)
    logits, thr = pl.pallas_call(
        functools.partial(_logits_body, scale=scale),
        grid=grid,
        in_specs=[
            pl.BlockSpec((1, TILE_N, D), lambda b, i: (b, i, 0)),
            pl.BlockSpec((D, D), lambda b, i: (0, 0)),
            pl.BlockSpec((D,), lambda b, i: (0,)),
            pl.BlockSpec((1, D, E), lambda b, i: (b, 0, 0)),
        ],
        out_specs=[
            pl.BlockSpec((TILE_N, E), lambda b, i: (b * (N // TILE_N) + i, 0)),
            pl.BlockSpec((8, TILE_N), lambda b, i: (b * (N // TILE_N) + i, 0)),
        ],
        out_shape=[
            jax.ShapeDtypeStruct((rows, E), jnp.float32),
            jax.ShapeDtypeStruct((rows // TILE_N * 8, TILE_N), jnp.float32),
        ],
    )(X, W_preT, b_pre, M_T)

    thr_flat = thr.reshape(rows // TILE_N, 8, TILE_N)[:, 0, :].reshape(rows)

    mesh = plsc.VectorSubcoreMesh(core_axis_name="c", subcore_axis_name="s")
    sc = pl.kernel(
        _sc_body,
        mesh=mesh,
        compiler_params=pltpu.CompilerParams(needs_layout_passes=False),
        out_type=[
            jax.ShapeDtypeStruct((rows * K_C,), jnp.int32),
            jax.ShapeDtypeStruct((rows * K_C,), jnp.float32),
            jax.ShapeDtypeStruct((rows,), jnp.int32),
        ],
        scratch_types=[
            pltpu.VMEM((16, E), jnp.float32),
            pltpu.VMEM((rows // NW,), jnp.float32),
            pltpu.VMEM((16 * CAPP,), jnp.float32),
            pltpu.VMEM((16 * CAPP,), jnp.int32),
            pltpu.VMEM((16 * K_C,), jnp.float32),
            pltpu.VMEM((16 * K_C,), jnp.int32),
            pltpu.VMEM((rows // NW,), jnp.int32),
        ],
    )
    idx_f, w_f, cnt_f = sc(logits, thr_flat)

    counts = cnt_f
    bad = jnp.any((counts < kE) | (counts > CAP))

    def fb(_):
        tv, ti = jax.lax.top_k(logits.reshape(B, N, E), kE)
        return ti.astype(jnp.int32), jax.nn.softmax(tv, axis=-1)

    def ok(_):
        return (idx_f.reshape(B, N, kE), w_f.reshape(B, N, kE))

    return jax.lax.cond(bad, fb, ok, None)
